# baseline (speedup 1.0000x reference)
import functools

import jax
import jax.numpy as jnp
from jax import lax
from jax.experimental import pallas as pl
from jax.experimental.pallas import tpu as pltpu
from jax.experimental.pallas import tpu_sc as plsc

_N = 10000
_E = 320000
_D = 128
_G = 64
_NP = 10240
_NW = 32
_CH = 128
_NCH = 80
_HCH = 40
_EP = _NW * _NCH * _CH
_BF = 2000
_BR = 2000


def _seg_sum_body(h, src3, dst3, out, src_v, dst_v, rows_a, rows_b, acc,
                  sem_a, sem_b):
    c = lax.axis_index("c")
    s = lax.axis_index("s")
    wid = s * 2 + c

    pltpu.sync_copy(src3.at[wid, pl.ds(0, _HCH)], src_v)
    pltpu.sync_copy(dst3.at[wid, pl.ds(0, _HCH)], dst_v)
    pltpu.async_copy(h.at[src_v.at[0]], rows_a, sem_a)

    zero16 = jnp.zeros((16,), jnp.float32)

    def zrow(r, carry):
        for j in range(8):
            rows_b[r, pl.ds(j * 16, 16)] = zero16
        return carry

    lax.fori_loop(0, _CH, zrow, 0)
    for k in range(_NP // 16 // _CH):
        pltpu.sync_copy(rows_b, acc.at[pl.ds(s * (_NP // 16) + k * _CH, _CH)])
    plsc.subcore_barrier()

    bufs = (rows_a, rows_b)
    sems = (sem_a, sem_b)

    def step(j, b, issue_next):
        if issue_next:
            pltpu.async_copy(h.at[src_v.at[j + 1]], bufs[1 - b], sems[1 - b])
        pltpu.make_async_copy(h.at[src_v.at[j]], bufs[b], sems[b]).wait()
        pltpu.sync_copy(bufs[b], acc.at[dst_v.at[j]], add=True)

    def outer(g, carry):
        step(g * 2, 0, True)
        step(g * 2 + 1, 1, True)
        return carry

    for p in range(_NCH // _HCH):
        if p > 0:
            pltpu.sync_copy(src3.at[wid, pl.ds(p * _HCH, _HCH)], src_v)
            pltpu.sync_copy(dst3.at[wid, pl.ds(p * _HCH, _HCH)], dst_v)
            pltpu.async_copy(h.at[src_v.at[0]], rows_a, sem_a)
        lax.fori_loop(0, _HCH // 2 - 1, outer, 0)
        step(_HCH - 2, 0, True)
        step(_HCH - 1, 1, False)

    plsc.subcore_barrier()
    pltpu.sync_copy(acc.at[pl.ds(s * (_NP // 16), _NP // 16)],
                    out.at[c, pl.ds(s * (_NP // 16), _NP // 16)])


@functools.cache
def _make_seg_sum():
    return pl.kernel(
        _seg_sum_body,
        out_type=jax.ShapeDtypeStruct((2, _NP, _D), jnp.float32),
        mesh=plsc.VectorSubcoreMesh(
            core_axis_name="c", subcore_axis_name="s", num_cores=2),
        scratch_types=[
            pltpu.VMEM((_HCH, _CH), jnp.int32),
            pltpu.VMEM((_HCH, _CH), jnp.int32),
            pltpu.VMEM((_CH, _D), jnp.float32),
            pltpu.VMEM((_CH, _D), jnp.float32),
            pltpu.VMEM_SHARED((_NP, _D), jnp.float32),
            pltpu.SemaphoreType.DMA,
            pltpu.SemaphoreType.DMA,
        ],
    )


def _seg_sum(h, src3, dst3):
    return _make_seg_sum()(h, src3, dst3)


def _mlp_body(scal_ref, h_ref, a_ref, w1_ref, b1_ref, w2_ref, b2_ref, o_ref,
              *, last):
    z = h_ref[...] * scal_ref[0, 0] + a_ref[0] + a_ref[1]
    z1 = jnp.dot(z, w1_ref[...], preferred_element_type=jnp.float32)
    z1 = jnp.maximum(z1 + b1_ref[...], 0.0)
    z2 = jnp.dot(z1, w2_ref[...], preferred_element_type=jnp.float32)
    z2 = z2 + b2_ref[...]
    if not last:
        z2 = jnp.maximum(z2, 0.0)
    o_ref[...] = z2


def _gin_mlp(scal, h, agg, w1, b1, w2, b2, last):
    return pl.pallas_call(
        functools.partial(_mlp_body, last=last),
        grid=(_N // _BR,),
        in_specs=[
            pl.BlockSpec(memory_space=pltpu.SMEM),
            pl.BlockSpec((_BR, _D), lambda i: (i, 0)),
            pl.BlockSpec((2, _BR, _D), lambda i: (0, i, 0)),
            pl.BlockSpec((_D, 2 * _D), lambda i: (0, 0)),
            pl.BlockSpec((1, 2 * _D), lambda i: (0, 0)),
            pl.BlockSpec((2 * _D, _D), lambda i: (0, 0)),
            pl.BlockSpec((1, _D), lambda i: (0, 0)),
        ],
        out_specs=pl.BlockSpec((_BR, _D), lambda i: (i, 0)),
        out_shape=jax.ShapeDtypeStruct((_N, _D), jnp.float32),
    )(scal, h, agg, w1, b1, w2, b2)


def _final_body(h_ref, bt_ref, nw_ref, nb_ref, pw_ref, pb_ref, o_ref,
                acc, cnt):
    i = pl.program_id(0)

    @pl.when(i == 0)
    def _init():
        acc[...] = jnp.zeros_like(acc)
        cnt[...] = jnp.zeros_like(cnt)

    t = jnp.dot(h_ref[...], nw_ref[...], preferred_element_type=jnp.float32)
    t = jnp.maximum(t + nb_ref[...], 0.0)
    b = bt_ref[0, 0, :]
    gids = lax.broadcasted_iota(jnp.int32, (_G, _BF), 0)
    oh = jnp.where(gids == b[None, :], 1.0, 0.0)
    acc[...] += jnp.dot(oh, t, preferred_element_type=jnp.float32)
    cnt[...] += jnp.dot(oh, jnp.ones((_BF, _D), jnp.float32),
                        preferred_element_type=jnp.float32)

    @pl.when(i == _N // _BF - 1)
    def _done():
        pooled = acc[...] / jnp.maximum(cnt[...], 1.0)
        o_ref[...] = (jnp.dot(pooled, pw_ref[...],
                              preferred_element_type=jnp.float32)
                      + pb_ref[...])


def _final(h, bt, n2n_w, n2n_b, pred_w, pred_b):
    return pl.pallas_call(
        _final_body,
        grid=(_N // _BF,),
        in_specs=[
            pl.BlockSpec((_BF, _D), lambda i: (i, 0)),
            pl.BlockSpec((1, 1, _BF), lambda i: (i, 0, 0)),
            pl.BlockSpec((_D, _D), lambda i: (0, 0)),
            pl.BlockSpec((1, _D), lambda i: (0, 0)),
            pl.BlockSpec((_D, _D), lambda i: (0, 0)),
            pl.BlockSpec((1, _D), lambda i: (0, 0)),
        ],
        out_specs=pl.BlockSpec((_G, _D), lambda i: (0, 0)),
        out_shape=jax.ShapeDtypeStruct((_G, _D), jnp.float32),
        scratch_shapes=[
            pltpu.VMEM((_G, _D), jnp.float32),
            pltpu.VMEM((_G, _D), jnp.float32),
        ],
    )(h, bt, n2n_w, n2n_b, pred_w, pred_b)


def kernel(x, edge_index, batch, gin_w1, gin_b1, gin_w2, gin_b2, gin_eps,
           n2n_w, n2n_b, pred_w, pred_b):
    base_src = (jnp.arange(_EP, dtype=jnp.int32) * 13) % _N
    base_dst = _N + (jnp.arange(_EP, dtype=jnp.int32) % (_NP - _N))
    src3 = lax.dynamic_update_slice(
        base_src, edge_index[0], (0,)).reshape(_NW, _NCH, _CH)
    dst3 = lax.dynamic_update_slice(
        base_dst, edge_index[1], (0,)).reshape(_NW, _NCH, _CH)
    h = x

    n_layers = gin_w1.shape[0]
    for l in range(n_layers):
        agg = _seg_sum(h, src3, dst3)
        scal = (1.0 + gin_eps[l]).reshape(1, 1)
        h = _gin_mlp(scal, h, agg, gin_w1[l], gin_b1[l].reshape(1, -1),
                     gin_w2[l], gin_b2[l].reshape(1, -1),
                     last=(l == n_layers - 1))

    bt = batch.reshape(_N // _BF, 1, _BF)
    return _final(h, bt, n2n_w, n2n_b.reshape(1, -1),
                  pred_w, pred_b.reshape(1, -1))


# speedup vs baseline: 1.0211x; 1.0211x over previous
import functools

import jax
import jax.numpy as jnp
import numpy as np
from jax import lax
from jax.experimental import pallas as pl
from jax.experimental.pallas import tpu as pltpu
from jax.experimental.pallas import tpu_sc as plsc

_N = 10000
_E = 320000
_D = 128
_G = 64
_NP = 10240
_NW = 32
_CH = 128
_NCH = 80
_HCH = 40
_EP = _NW * _NCH * _CH
_BF = 2000
_BR = 2000


def _seg_sum_body(h, src3, dst3, out, src_v, dst_v, rows_a, rows_b, acc,
                  sem_a, sem_b):
    c = lax.axis_index("c")
    s = lax.axis_index("s")
    wid = s * 2 + c

    pltpu.sync_copy(src3.at[wid, pl.ds(0, _HCH)], src_v)
    pltpu.sync_copy(dst3.at[wid, pl.ds(0, _HCH)], dst_v)
    pltpu.async_copy(h.at[src_v.at[0]], rows_a, sem_a)

    zero16 = jnp.zeros((16,), jnp.float32)

    def zrow(r, carry):
        for j in range(8):
            rows_b[r, pl.ds(j * 16, 16)] = zero16
        return carry

    lax.fori_loop(0, _CH, zrow, 0)
    for k in range(_NP // 16 // _CH):
        pltpu.sync_copy(rows_b, acc.at[pl.ds(s * (_NP // 16) + k * _CH, _CH)])
    plsc.subcore_barrier()

    bufs = (rows_a, rows_b)
    sems = (sem_a, sem_b)

    def step(j, b, issue_next):
        if issue_next:
            pltpu.async_copy(h.at[src_v.at[j + 1]], bufs[1 - b], sems[1 - b])
        pltpu.make_async_copy(h.at[src_v.at[j]], bufs[b], sems[b]).wait()
        pltpu.sync_copy(bufs[b], acc.at[dst_v.at[j]], add=True)

    def outer(g, carry):
        step(g * 2, 0, True)
        step(g * 2 + 1, 1, True)
        return carry

    for p in range(_NCH // _HCH):
        if p > 0:
            pltpu.sync_copy(src3.at[wid, pl.ds(p * _HCH, _HCH)], src_v)
            pltpu.sync_copy(dst3.at[wid, pl.ds(p * _HCH, _HCH)], dst_v)
            pltpu.async_copy(h.at[src_v.at[0]], rows_a, sem_a)
        lax.fori_loop(0, _HCH // 2 - 1, outer, 0)
        step(_HCH - 2, 0, True)
        step(_HCH - 1, 1, False)

    plsc.subcore_barrier()
    pltpu.sync_copy(acc.at[pl.ds(s * (_NP // 16), _NP // 16)],
                    out.at[c, pl.ds(s * (_NP // 16), _NP // 16)])


@functools.cache
def _make_seg_sum():
    return pl.kernel(
        _seg_sum_body,
        out_type=jax.ShapeDtypeStruct((2, _NP, _D), jnp.float32),
        mesh=plsc.VectorSubcoreMesh(
            core_axis_name="c", subcore_axis_name="s", num_cores=2),
        scratch_types=[
            pltpu.VMEM((_HCH, _CH), jnp.int32),
            pltpu.VMEM((_HCH, _CH), jnp.int32),
            pltpu.VMEM((_CH, _D), jnp.float32),
            pltpu.VMEM((_CH, _D), jnp.float32),
            pltpu.VMEM_SHARED((_NP, _D), jnp.float32),
            pltpu.SemaphoreType.DMA,
            pltpu.SemaphoreType.DMA,
        ],
    )


def _seg_sum(h, src3, dst3):
    return _make_seg_sum()(h, src3, dst3)


def _mlp_body(scal_ref, h_ref, a_ref, w1_ref, b1_ref, w2_ref, b2_ref, o_ref,
              *, last):
    z = h_ref[...] * scal_ref[0, 0] + a_ref[0] + a_ref[1]
    z1 = jnp.dot(z, w1_ref[...], preferred_element_type=jnp.float32)
    z1 = jnp.maximum(z1 + b1_ref[...], 0.0)
    z2 = jnp.dot(z1, w2_ref[...], preferred_element_type=jnp.float32)
    z2 = z2 + b2_ref[...]
    if not last:
        z2 = jnp.maximum(z2, 0.0)
    o_ref[...] = z2


def _gin_mlp(scal, h, agg, w1, b1, w2, b2, last):
    return pl.pallas_call(
        functools.partial(_mlp_body, last=last),
        grid=(_N // _BR,),
        in_specs=[
            pl.BlockSpec(memory_space=pltpu.SMEM),
            pl.BlockSpec((_BR, _D), lambda i: (i, 0)),
            pl.BlockSpec((2, _BR, _D), lambda i: (0, i, 0)),
            pl.BlockSpec((_D, 2 * _D), lambda i: (0, 0)),
            pl.BlockSpec((1, 2 * _D), lambda i: (0, 0)),
            pl.BlockSpec((2 * _D, _D), lambda i: (0, 0)),
            pl.BlockSpec((1, _D), lambda i: (0, 0)),
        ],
        out_specs=pl.BlockSpec((_BR, _D), lambda i: (i, 0)),
        out_shape=jax.ShapeDtypeStruct((_N, _D), jnp.float32),
    )(scal, h, agg, w1, b1, w2, b2)


def _final_body(h_ref, bt_ref, nw_ref, nb_ref, pw_ref, pb_ref, o_ref,
                acc, cnt):
    i = pl.program_id(0)

    @pl.when(i == 0)
    def _init():
        acc[...] = jnp.zeros_like(acc)
        cnt[...] = jnp.zeros_like(cnt)

    t = jnp.dot(h_ref[...], nw_ref[...], preferred_element_type=jnp.float32)
    t = jnp.maximum(t + nb_ref[...], 0.0)
    b = bt_ref[0, 0, :]
    gids = lax.broadcasted_iota(jnp.int32, (_G, _BF), 0)
    oh = jnp.where(gids == b[None, :], 1.0, 0.0)
    acc[...] += jnp.dot(oh, t, preferred_element_type=jnp.float32)
    cnt[...] += jnp.dot(oh, jnp.ones((_BF, _D), jnp.float32),
                        preferred_element_type=jnp.float32)

    @pl.when(i == _N // _BF - 1)
    def _done():
        pooled = acc[...] / jnp.maximum(cnt[...], 1.0)
        o_ref[...] = (jnp.dot(pooled, pw_ref[...],
                              preferred_element_type=jnp.float32)
                      + pb_ref[...])


def _final(h, bt, n2n_w, n2n_b, pred_w, pred_b):
    return pl.pallas_call(
        _final_body,
        grid=(_N // _BF,),
        in_specs=[
            pl.BlockSpec((_BF, _D), lambda i: (i, 0)),
            pl.BlockSpec((1, 1, _BF), lambda i: (i, 0, 0)),
            pl.BlockSpec((_D, _D), lambda i: (0, 0)),
            pl.BlockSpec((1, _D), lambda i: (0, 0)),
            pl.BlockSpec((_D, _D), lambda i: (0, 0)),
            pl.BlockSpec((1, _D), lambda i: (0, 0)),
        ],
        out_specs=pl.BlockSpec((_G, _D), lambda i: (0, 0)),
        out_shape=jax.ShapeDtypeStruct((_G, _D), jnp.float32),
        scratch_shapes=[
            pltpu.VMEM((_G, _D), jnp.float32),
            pltpu.VMEM((_G, _D), jnp.float32),
        ],
    )(h, bt, n2n_w, n2n_b, pred_w, pred_b)


def kernel(x, edge_index, batch, gin_w1, gin_b1, gin_w2, gin_b2, gin_eps,
           n2n_w, n2n_b, pred_w, pred_b):
    base_src = jnp.asarray((np.arange(_EP) * 13 % _N).astype(np.int32))
    base_dst = jnp.asarray((_N + np.arange(_EP) % (_NP - _N)).astype(np.int32))
    src3 = lax.dynamic_update_slice(
        base_src, edge_index[0], (0,)).reshape(_NW, _NCH, _CH)
    dst3 = lax.dynamic_update_slice(
        base_dst, edge_index[1], (0,)).reshape(_NW, _NCH, _CH)
    h = x

    n_layers = gin_w1.shape[0]
    for l in range(n_layers):
        agg = _seg_sum(h, src3, dst3)
        scal = (1.0 + gin_eps[l]).reshape(1, 1)
        h = _gin_mlp(scal, h, agg, gin_w1[l], gin_b1[l].reshape(1, -1),
                     gin_w2[l], gin_b2[l].reshape(1, -1),
                     last=(l == n_layers - 1))

    bt = batch.reshape(_N // _BF, 1, _BF)
    return _final(h, bt, n2n_w, n2n_b.reshape(1, -1),
                  pred_w, pred_b.reshape(1, -1))


# speedup vs baseline: 1.0514x; 1.0297x over previous
import functools

import jax
import jax.numpy as jnp
import numpy as np
from jax import lax
from jax.experimental import pallas as pl
from jax.experimental.pallas import tpu as pltpu
from jax.experimental.pallas import tpu_sc as plsc

_N = 10000
_E = 320000
_D = 128
_G = 64
_NP = 10240
_NW = 32
_CH = 128
_NCH = 80
_HCH = 40
_EP = _NW * _NCH * _CH
_BF = 2000
_BR = 2000


def _seg_sum_body(h, src3, dst3, out, src_v, dst_v, rows_a, rows_b, acc,
                  sem_a, sem_b):
    c = lax.axis_index("c")
    s = lax.axis_index("s")
    wid = s * 2 + c

    pltpu.sync_copy(src3.at[wid, pl.ds(0, _HCH)], src_v)
    pltpu.sync_copy(dst3.at[wid, pl.ds(0, _HCH)], dst_v)
    pltpu.async_copy(h.at[src_v.at[0]], rows_a, sem_a)

    zero16 = jnp.zeros((16,), jnp.float32)

    def zrow(r, carry):
        for j in range(8):
            rows_b[r, pl.ds(j * 16, 16)] = zero16
        return carry

    lax.fori_loop(0, _CH, zrow, 0)
    for k in range(_NP // 16 // _CH):
        pltpu.sync_copy(rows_b, acc.at[pl.ds(s * (_NP // 16) + k * _CH, _CH)])
    plsc.subcore_barrier()

    bufs = (rows_a, rows_b)
    sems = (sem_a, sem_b)

    def step(j, b, issue_next):
        if issue_next:
            pltpu.async_copy(h.at[src_v.at[j + 1]], bufs[1 - b], sems[1 - b])
        pltpu.make_async_copy(h.at[src_v.at[j]], bufs[b], sems[b]).wait()
        pltpu.sync_copy(bufs[b], acc.at[dst_v.at[j]], add=True)

    def outer(g, carry):
        step(g * 2, 0, True)
        step(g * 2 + 1, 1, True)
        return carry

    for p in range(_NCH // _HCH):
        if p > 0:
            pltpu.sync_copy(src3.at[wid, pl.ds(p * _HCH, _HCH)], src_v)
            pltpu.sync_copy(dst3.at[wid, pl.ds(p * _HCH, _HCH)], dst_v)
            pltpu.async_copy(h.at[src_v.at[0]], rows_a, sem_a)
        lax.fori_loop(0, _HCH // 2 - 1, outer, 0)
        step(_HCH - 2, 0, True)
        step(_HCH - 1, 1, False)

    plsc.subcore_barrier()
    pltpu.sync_copy(acc.at[pl.ds(s * (_NP // 16), _NP // 16)],
                    out.at[c, pl.ds(s * (_NP // 16), _NP // 16)])


@functools.cache
def _make_seg_sum():
    return pl.kernel(
        _seg_sum_body,
        out_type=jax.ShapeDtypeStruct((2, _NP, _D), jnp.float32),
        mesh=plsc.VectorSubcoreMesh(
            core_axis_name="c", subcore_axis_name="s", num_cores=2),
        scratch_types=[
            pltpu.VMEM((_HCH, _CH), jnp.int32),
            pltpu.VMEM((_HCH, _CH), jnp.int32),
            pltpu.VMEM((_CH, _D), jnp.float32),
            pltpu.VMEM((_CH, _D), jnp.float32),
            pltpu.VMEM_SHARED((_NP, _D), jnp.float32),
            pltpu.SemaphoreType.DMA,
            pltpu.SemaphoreType.DMA,
        ],
    )


def _seg_sum(h, src3, dst3):
    return _make_seg_sum()(h, src3, dst3)


def _mlp_body(scal_ref, h_ref, a_ref, w1_ref, b1_ref, w2_ref, b2_ref, o_ref,
              *, last):
    z = h_ref[...] * scal_ref[0, 0] + a_ref[0] + a_ref[1]
    z1 = jnp.dot(z, w1_ref[...], preferred_element_type=jnp.float32)
    z1 = jnp.maximum(z1 + b1_ref[...], 0.0)
    z2 = jnp.dot(z1, w2_ref[...], preferred_element_type=jnp.float32)
    z2 = z2 + b2_ref[...]
    if not last:
        z2 = jnp.maximum(z2, 0.0)
    o_ref[...] = z2


def _gin_mlp(scal, h, agg, w1, b1, w2, b2, last):
    return pl.pallas_call(
        functools.partial(_mlp_body, last=last),
        grid=(_N // _BR,),
        in_specs=[
            pl.BlockSpec(memory_space=pltpu.SMEM),
            pl.BlockSpec((_BR, _D), lambda i: (i, 0)),
            pl.BlockSpec((2, _BR, _D), lambda i: (0, i, 0)),
            pl.BlockSpec((_D, 2 * _D), lambda i: (0, 0)),
            pl.BlockSpec((1, 2 * _D), lambda i: (0, 0)),
            pl.BlockSpec((2 * _D, _D), lambda i: (0, 0)),
            pl.BlockSpec((1, _D), lambda i: (0, 0)),
        ],
        out_specs=pl.BlockSpec((_BR, _D), lambda i: (i, 0)),
        out_shape=jax.ShapeDtypeStruct((_N, _D), jnp.float32),
    )(scal, h, agg, w1, b1, w2, b2)


def _final_body(h_ref, bt_ref, nw_ref, nb_ref, pw_ref, pb_ref, o_ref,
                acc, cnt):
    i = pl.program_id(0)

    @pl.when(i == 0)
    def _init():
        acc[...] = jnp.zeros_like(acc)
        cnt[...] = jnp.zeros_like(cnt)

    t = jnp.dot(h_ref[...], nw_ref[...], preferred_element_type=jnp.float32)
    t = jnp.maximum(t + nb_ref[...], 0.0)
    b = bt_ref[0, 0, :]
    gids = lax.broadcasted_iota(jnp.int32, (_G, _BF), 0)
    oh = jnp.where(gids == b[None, :], 1.0, 0.0)
    acc[...] += jnp.dot(oh, t, preferred_element_type=jnp.float32)
    cnt[...] += jnp.dot(oh, jnp.ones((_BF, _D), jnp.float32),
                        preferred_element_type=jnp.float32)

    @pl.when(i == _N // _BF - 1)
    def _done():
        pooled = acc[...] / jnp.maximum(cnt[...], 1.0)
        o_ref[...] = (jnp.dot(pooled, pw_ref[...],
                              preferred_element_type=jnp.float32)
                      + pb_ref[...])


def _final(h, bt, n2n_w, n2n_b, pred_w, pred_b):
    return pl.pallas_call(
        _final_body,
        grid=(_N // _BF,),
        in_specs=[
            pl.BlockSpec((_BF, _D), lambda i: (i, 0)),
            pl.BlockSpec((1, 1, _BF), lambda i: (i, 0, 0)),
            pl.BlockSpec((_D, _D), lambda i: (0, 0)),
            pl.BlockSpec((1, _D), lambda i: (0, 0)),
            pl.BlockSpec((_D, _D), lambda i: (0, 0)),
            pl.BlockSpec((1, _D), lambda i: (0, 0)),
        ],
        out_specs=pl.BlockSpec((_G, _D), lambda i: (0, 0)),
        out_shape=jax.ShapeDtypeStruct((_G, _D), jnp.float32),
        scratch_shapes=[
            pltpu.VMEM((_G, _D), jnp.float32),
            pltpu.VMEM((_G, _D), jnp.float32),
        ],
    )(h, bt, n2n_w, n2n_b, pred_w, pred_b)


def kernel(x, edge_index, batch, gin_w1, gin_b1, gin_w2, gin_b2, gin_eps,
           n2n_w, n2n_b, pred_w, pred_b):
    npad = (_EP - _E) // _CH
    pad_src = jnp.asarray(
        (np.arange(_EP - _E) & 8191).astype(np.int32).reshape(npad, _CH))
    pad_dst = jnp.asarray(
        (_N + (np.arange(_EP - _E) & 127)).astype(np.int32).reshape(npad, _CH))
    src3 = jnp.concatenate(
        [edge_index[0].reshape(_E // _CH, _CH), pad_src]).reshape(
            _NW, _NCH, _CH)
    dst3 = jnp.concatenate(
        [edge_index[1].reshape(_E // _CH, _CH), pad_dst]).reshape(
            _NW, _NCH, _CH)
    h = x

    n_layers = gin_w1.shape[0]
    for l in range(n_layers):
        agg = _seg_sum(h, src3, dst3)
        scal = (1.0 + gin_eps[l]).reshape(1, 1)
        h = _gin_mlp(scal, h, agg, gin_w1[l], gin_b1[l].reshape(1, -1),
                     gin_w2[l], gin_b2[l].reshape(1, -1),
                     last=(l == n_layers - 1))

    bt = batch.reshape(_N // _BF, 1, _BF)
    return _final(h, bt, n2n_w, n2n_b.reshape(1, -1),
                  pred_w, pred_b.reshape(1, -1))
